# baseline (speedup 1.0000x reference)
import jax
import jax.numpy as jnp
from jax.experimental import pallas as pl

_GRID = 16
_NODE_STEPS = 10


def _copy_body(node_ref, edge_ref, node_out_ref, edge_out_ref):
    edge_out_ref[...] = edge_ref[...]

    @pl.when(pl.program_id(0) < _NODE_STEPS)
    def _():
        node_out_ref[...] = node_ref[...]


def kernel(node_feats, edge_index, edge_attr):
    n_nodes, d_feat = node_feats.shape
    n_edges, d_edge = edge_attr.shape
    nb = n_nodes // _NODE_STEPS
    eb = n_edges // _GRID

    def node_idx(i):
        return (jnp.minimum(i, _NODE_STEPS - 1), 0)

    node_out, edge_out = pl.pallas_call(
        _copy_body,
        grid=(_GRID,),
        in_specs=[
            pl.BlockSpec((nb, d_feat), node_idx),
            pl.BlockSpec((eb, d_edge), lambda i: (i, 0)),
        ],
        out_specs=[
            pl.BlockSpec((nb, d_feat), node_idx),
            pl.BlockSpec((eb, d_edge), lambda i: (i, 0)),
        ],
        out_shape=[
            jax.ShapeDtypeStruct((n_nodes, d_feat), node_feats.dtype),
            jax.ShapeDtypeStruct((n_edges, d_edge), edge_attr.dtype),
        ],
    )(node_feats, edge_attr)
    return (node_out, edge_out)


# speedup vs baseline: 1.0025x; 1.0025x over previous
import jax
import jax.numpy as jnp
from jax.experimental import pallas as pl

_GRID = 20
_NODE_STEPS = 10


def _copy_body(node_ref, edge_ref, node_out_ref, edge_out_ref):
    edge_out_ref[...] = edge_ref[...]

    @pl.when(pl.program_id(0) < _NODE_STEPS)
    def _():
        node_out_ref[...] = node_ref[...]


def kernel(node_feats, edge_index, edge_attr):
    n_nodes, d_feat = node_feats.shape
    n_edges, d_edge = edge_attr.shape
    nb = n_nodes // _NODE_STEPS
    eb = n_edges // _GRID

    def node_idx(i):
        return (jnp.minimum(i, _NODE_STEPS - 1), 0)

    node_out, edge_out = pl.pallas_call(
        _copy_body,
        grid=(_GRID,),
        in_specs=[
            pl.BlockSpec((nb, d_feat), node_idx),
            pl.BlockSpec((eb, d_edge), lambda i: (i, 0)),
        ],
        out_specs=[
            pl.BlockSpec((nb, d_feat), node_idx),
            pl.BlockSpec((eb, d_edge), lambda i: (i, 0)),
        ],
        out_shape=[
            jax.ShapeDtypeStruct((n_nodes, d_feat), node_feats.dtype),
            jax.ShapeDtypeStruct((n_edges, d_edge), edge_attr.dtype),
        ],
    )(node_feats, edge_attr)
    return (node_out, edge_out)
